# Initial kernel scaffold; baseline (speedup 1.0000x reference)
import jax
import jax.numpy as jnp
from jax.experimental import pallas as pl


def kernel(init_embed_real, init_embed_imag, init_rel_real, init_rel_imag, im_proj, W_ent, W_rel, edge_index, edge_type, sub, rel):
    raise NotImplementedError("write your pallas kernel here")


# speedup vs baseline: 5.1013x; 5.1013x over previous
import functools

import jax
import jax.numpy as jnp
from jax import lax
from jax.experimental import pallas as pl
from jax.experimental.pallas import tpu as pltpu
from jax.experimental.pallas import tpu_sc as plsc

NUM_ENT_K = 10000
NUM_RELROWS_K = 400
N_EDGES_K = 320000
DIM_K = 128
HALF_K = 64
BATCH_K = 4096
LANES = 16
NCORES = 2
NSUB = 16
CHUNK = 128
NCHUNKS = N_EDGES_K // CHUNK
STRIPE = 640
BB = 80

_f32 = jnp.float32
_i32 = jnp.int32


def _proj_body(ei_ref, ri_ref, p_ref, eo_ref, ro_ref):
    p = p_ref[...]
    eo_ref[...] = jnp.dot(ei_ref[...], p, preferred_element_type=_f32)
    ro_ref[...] = jnp.dot(ri_ref[...], p, preferred_element_type=_f32)


def _node_body(agg_r_ref, agg_i_ref, deg_ref, rel_r_ref, rel_i_ref,
               we_ref, wr_ref, or_ref, oi_ref, ror_ref, roi_ref):
    deg = deg_ref[0:NUM_ENT_K, 0:1] + deg_ref[NUM_ENT_K:2 * NUM_ENT_K, 0:1]
    inv = jnp.where(deg == 0.0, 1.0, 1.0 / deg)
    w0 = we_ref[0:HALF_K, :]
    w1 = we_ref[HALF_K:DIM_K, :]
    ar0 = agg_r_ref[0:NUM_ENT_K, :] * inv
    ar1 = agg_r_ref[NUM_ENT_K:2 * NUM_ENT_K, :] * inv
    ai0 = agg_i_ref[0:NUM_ENT_K, :] * inv
    ai1 = agg_i_ref[NUM_ENT_K:2 * NUM_ENT_K, :] * inv
    or_ref[...] = jnp.tanh(jnp.dot(ar0, w0, preferred_element_type=_f32)
                           + jnp.dot(ar1, w1, preferred_element_type=_f32))
    oi_ref[...] = jnp.tanh(jnp.dot(ai0, w0, preferred_element_type=_f32)
                           + jnp.dot(ai1, w1, preferred_element_type=_f32))
    wr = wr_ref[...]
    ror_ref[...] = jnp.dot(rel_r_ref[...], wr, preferred_element_type=_f32)
    roi_ref[...] = jnp.dot(rel_i_ref[...], wr, preferred_element_type=_f32)


_MESH = plsc.VectorSubcoreMesh(core_axis_name="c", subcore_axis_name="s",
                               num_cores=NCORES, num_subcores=NSUB)


def _edge_body(ent_r_hbm, ent_i_hbm, rel_r_hbm, rel_i_hbm,
               src_hbm, dst_hbm, et_hbm,
               agg_r_out, agg_i_out,
               idx_src, idx_dst, idx_et,
               h_r, h_i, r_r, r_i,
               zbuf,
               agg_r_sp, agg_i_sp,
               sem0, sem1, sem2, sem3):
    c = lax.axis_index("c")
    s = lax.axis_index("s")
    zero16 = jnp.zeros((LANES,), _f32)

    def _z_zbuf(e, carry):
        for j in range(HALF_K // LANES):
            zbuf[e, pl.ds(j * LANES, LANES)] = zero16
        return carry
    lax.fori_loop(0, BB, _z_zbuf, 0)

    base = s * STRIPE
    nb = jnp.where(s == NSUB - 1, (NUM_ENT_K - (NSUB - 1) * STRIPE) // BB,
                   STRIPE // BB)

    def _z_sp(b, carry):
        off = base + b * BB
        pltpu.sync_copy(zbuf, agg_r_sp.at[pl.ds(off, BB)])
        pltpu.sync_copy(zbuf, agg_i_sp.at[pl.ds(off, BB)])
        return carry
    lax.fori_loop(0, nb, _z_sp, 0)
    plsc.subcore_barrier()

    ent_off = c * NUM_ENT_K
    rel_off = c * NUM_RELROWS_K
    nq = jnp.where(s < NCHUNKS - (NCHUNKS // NSUB) * NSUB,
                   NCHUNKS // NSUB + 1, NCHUNKS // NSUB)

    def _chunk(q, carry):
        eoff = (q * NSUB + s) * CHUNK
        pltpu.sync_copy(src_hbm.at[pl.ds(eoff, CHUNK)], idx_src)
        pltpu.sync_copy(dst_hbm.at[pl.ds(eoff, CHUNK)], idx_dst)
        pltpu.sync_copy(et_hbm.at[pl.ds(eoff, CHUNK)], idx_et)

        def _shift(i, carry2):
            sl = pl.ds(i * LANES, LANES)
            idx_src[sl] = idx_src[sl] + ent_off
            idx_et[sl] = idx_et[sl] + rel_off
            return carry2
        lax.fori_loop(0, CHUNK // LANES, _shift, 0)

        cp0 = pltpu.async_copy(ent_r_hbm.at[idx_src], h_r, sem0)
        cp1 = pltpu.async_copy(ent_i_hbm.at[idx_src], h_i, sem1)
        cp2 = pltpu.async_copy(rel_r_hbm.at[idx_et], r_r, sem2)
        cp3 = pltpu.async_copy(rel_i_hbm.at[idx_et], r_i, sem3)
        cp0.wait()
        cp1.wait()
        cp2.wait()
        cp3.wait()

        def _row(e, carry2):
            for j in range(HALF_K // LANES):
                sl = pl.ds(j * LANES, LANES)
                hr = h_r[e, sl]
                hi = h_i[e, sl]
                rr = r_r[e, sl]
                ri = r_i[e, sl]
                h_r[e, sl] = hr * rr - hi * ri
                h_i[e, sl] = hr * ri + hi * rr
            return carry2
        lax.fori_loop(0, CHUNK, _row, 0)

        pltpu.sync_copy(h_r, agg_r_sp.at[idx_dst], add=True)
        pltpu.sync_copy(h_i, agg_i_sp.at[idx_dst], add=True)
        return carry
    lax.fori_loop(0, nq, _chunk, 0)

    plsc.subcore_barrier()

    def _wb(b, carry):
        off = base + b * BB
        pltpu.sync_copy(agg_r_sp.at[pl.ds(off, BB)], zbuf)
        pltpu.sync_copy(zbuf, agg_r_out.at[pl.ds(ent_off + off, BB)])
        pltpu.sync_copy(agg_i_sp.at[pl.ds(off, BB)], zbuf)
        pltpu.sync_copy(zbuf, agg_i_out.at[pl.ds(ent_off + off, BB)])
        return carry
    lax.fori_loop(0, nb, _wb, 0)


_edge_kernel = functools.partial(
    pl.kernel,
    out_type=(
        jax.ShapeDtypeStruct((2 * NUM_ENT_K, HALF_K), _f32),
        jax.ShapeDtypeStruct((2 * NUM_ENT_K, HALF_K), _f32),
    ),
    mesh=_MESH,
    scratch_types=[
        pltpu.VMEM((CHUNK,), _i32),
        pltpu.VMEM((CHUNK,), _i32),
        pltpu.VMEM((CHUNK,), _i32),
        pltpu.VMEM((CHUNK, HALF_K), _f32),
        pltpu.VMEM((CHUNK, HALF_K), _f32),
        pltpu.VMEM((CHUNK, HALF_K), _f32),
        pltpu.VMEM((CHUNK, HALF_K), _f32),
        pltpu.VMEM((BB, HALF_K), _f32),
        pltpu.VMEM_SHARED((NUM_ENT_K, HALF_K), _f32),
        pltpu.VMEM_SHARED((NUM_ENT_K, HALF_K), _f32),
        pltpu.SemaphoreType.DMA,
        pltpu.SemaphoreType.DMA,
        pltpu.SemaphoreType.DMA,
        pltpu.SemaphoreType.DMA,
    ],
    compiler_params=pltpu.CompilerParams(use_tc_tiling_on_sc=False),
)(_edge_body)


_EDGES_PER_CORE = N_EDGES_K // NCORES
_DCHUNKS = _EDGES_PER_CORE // CHUNK


def _deg_body(dst_hbm, deg_out, idx_dst, ones_v, zdeg, deg_sp, sem0):
    c = lax.axis_index("c")
    s = lax.axis_index("s")
    zero16 = jnp.zeros((LANES,), _f32)
    one16 = jnp.ones((LANES,), _f32)

    def _fill_row(e, carry):
        ones_v[e, :] = one16
        return carry
    lax.fori_loop(0, CHUNK, _fill_row, 0)

    def _z_zdeg(e, carry):
        zdeg[e, :] = zero16
        return carry
    lax.fori_loop(0, BB, _z_zdeg, 0)

    base = s * STRIPE
    nb = jnp.where(s == NSUB - 1, (NUM_ENT_K - (NSUB - 1) * STRIPE) // BB,
                   STRIPE // BB)

    def _z_sp(b, carry):
        pltpu.sync_copy(zdeg, deg_sp.at[pl.ds(base + b * BB, BB)])
        return carry
    lax.fori_loop(0, nb, _z_sp, 0)
    plsc.subcore_barrier()

    nq = jnp.where(s < _DCHUNKS - (_DCHUNKS // NSUB) * NSUB,
                   _DCHUNKS // NSUB + 1, _DCHUNKS // NSUB)

    def _chunk(q, carry):
        eoff = c * _EDGES_PER_CORE + (q * NSUB + s) * CHUNK
        pltpu.sync_copy(dst_hbm.at[pl.ds(eoff, CHUNK)], idx_dst)
        pltpu.sync_copy(ones_v, deg_sp.at[idx_dst], add=True)
        return carry
    lax.fori_loop(0, nq, _chunk, 0)

    plsc.subcore_barrier()

    def _wb(b, carry):
        off = base + b * BB
        pltpu.sync_copy(deg_sp.at[pl.ds(off, BB)], zdeg)
        pltpu.sync_copy(zdeg, deg_out.at[pl.ds(c * NUM_ENT_K + off, BB)])
        return carry
    lax.fori_loop(0, nb, _wb, 0)


_deg_kernel = functools.partial(
    pl.kernel,
    out_type=jax.ShapeDtypeStruct((2 * NUM_ENT_K, LANES), _f32),
    mesh=_MESH,
    scratch_types=[
        pltpu.VMEM((CHUNK,), _i32),
        pltpu.VMEM((CHUNK, LANES), _f32),
        pltpu.VMEM((BB, LANES), _f32),
        pltpu.VMEM_SHARED((NUM_ENT_K, LANES), _f32),
        pltpu.SemaphoreType.DMA,
    ],
    compiler_params=pltpu.CompilerParams(use_tc_tiling_on_sc=False),
)(_deg_body)


def _gather_body(out_r_hbm, out_i_hbm, ror_hbm, roi_hbm, sub_hbm, rel_hbm,
                 ser_out, sei_out, rer_out, rei_out,
                 idx_v, buf, sem):
    c = lax.axis_index("c")
    s = lax.axis_index("s")
    wid = s * NCORES + c
    per = BATCH_K // (NCORES * NSUB)
    base = wid * per
    pltpu.sync_copy(sub_hbm.at[pl.ds(base, per)], idx_v)
    pltpu.async_copy(out_r_hbm.at[idx_v], buf, sem).wait()
    pltpu.sync_copy(buf, ser_out.at[pl.ds(base, per)])
    pltpu.async_copy(out_i_hbm.at[idx_v], buf, sem).wait()
    pltpu.sync_copy(buf, sei_out.at[pl.ds(base, per)])
    pltpu.sync_copy(rel_hbm.at[pl.ds(base, per)], idx_v)
    pltpu.async_copy(ror_hbm.at[idx_v], buf, sem).wait()
    pltpu.sync_copy(buf, rer_out.at[pl.ds(base, per)])
    pltpu.async_copy(roi_hbm.at[idx_v], buf, sem).wait()
    pltpu.sync_copy(buf, rei_out.at[pl.ds(base, per)])


_gather_kernel = functools.partial(
    pl.kernel,
    out_type=(
        jax.ShapeDtypeStruct((BATCH_K, DIM_K), _f32),
        jax.ShapeDtypeStruct((BATCH_K, DIM_K), _f32),
        jax.ShapeDtypeStruct((BATCH_K, DIM_K), _f32),
        jax.ShapeDtypeStruct((BATCH_K, DIM_K), _f32),
    ),
    mesh=_MESH,
    scratch_types=[
        pltpu.VMEM((BATCH_K // (NCORES * NSUB),), _i32),
        pltpu.VMEM((BATCH_K // (NCORES * NSUB), DIM_K), _f32),
        pltpu.SemaphoreType.DMA,
    ],
)(_gather_body)


def kernel(init_embed_real, init_embed_imag, init_rel_real, init_rel_imag,
           im_proj, W_ent, W_rel, edge_index, edge_type, sub, rel):
    ent_i, rel_i = pl.pallas_call(
        _proj_body,
        out_shape=(
            jax.ShapeDtypeStruct((NUM_ENT_K, DIM_K), _f32),
            jax.ShapeDtypeStruct((NUM_RELROWS_K, DIM_K), _f32),
        ),
    )(init_embed_imag, init_rel_imag, im_proj)

    def _split(t):
        return jnp.concatenate([t[:, :HALF_K], t[:, HALF_K:]], axis=0)

    src = edge_index[0].astype(_i32)
    dst = edge_index[1].astype(_i32)
    et = edge_type.astype(_i32)

    deg16 = _deg_kernel(dst)
    agg_r2, agg_i2 = _edge_kernel(
        _split(init_embed_real), _split(ent_i),
        _split(init_rel_real), _split(rel_i),
        src, dst, et)

    out_r, out_i, rel_out_r, rel_out_i = pl.pallas_call(
        _node_body,
        out_shape=(
            jax.ShapeDtypeStruct((NUM_ENT_K, DIM_K), _f32),
            jax.ShapeDtypeStruct((NUM_ENT_K, DIM_K), _f32),
            jax.ShapeDtypeStruct((NUM_RELROWS_K, DIM_K), _f32),
            jax.ShapeDtypeStruct((NUM_RELROWS_K, DIM_K), _f32),
        ),
    )(agg_r2, agg_i2, deg16, init_rel_real, rel_i, W_ent, W_rel)

    sub_emb_r, sub_emb_i, rel_emb_r, rel_emb_i = _gather_kernel(
        out_r, out_i, rel_out_r, rel_out_i,
        sub.astype(_i32), rel.astype(_i32))

    return (sub_emb_r, sub_emb_i, rel_emb_r, rel_emb_i, out_r, out_i)


# speedup vs baseline: 6.5029x; 1.2748x over previous
import functools

import jax
import jax.numpy as jnp
from jax import lax
from jax.experimental import pallas as pl
from jax.experimental.pallas import tpu as pltpu
from jax.experimental.pallas import tpu_sc as plsc

NUM_ENT_K = 10000
NUM_RELROWS_K = 400
N_EDGES_K = 320000
DIM_K = 128
HALF_K = 64
BATCH_K = 4096
LANES = 16
NCORES = 2
NSUB = 16
CHUNK = 80
NQ = N_EDGES_K // CHUNK // NSUB
STRIPE = 640
BB = 40

_f32 = jnp.float32
_i32 = jnp.int32


def _proj_body(er_ref, ei_ref, rr_ref, ri_ref, p_ref,
               ent2_ref, rel2_ref, rip_ref):
    p = p_ref[...]
    eip = jnp.dot(ei_ref[...], p, preferred_element_type=_f32)
    rip = jnp.dot(ri_ref[...], p, preferred_element_type=_f32)
    er = er_ref[...]
    rr = rr_ref[...]
    ent2_ref[0:NUM_ENT_K, 0:HALF_K] = er[:, 0:HALF_K]
    ent2_ref[0:NUM_ENT_K, HALF_K:DIM_K] = eip[:, 0:HALF_K]
    ent2_ref[NUM_ENT_K:2 * NUM_ENT_K, 0:HALF_K] = er[:, HALF_K:DIM_K]
    ent2_ref[NUM_ENT_K:2 * NUM_ENT_K, HALF_K:DIM_K] = eip[:, HALF_K:DIM_K]
    rel2_ref[0:NUM_RELROWS_K, 0:HALF_K] = rr[:, 0:HALF_K]
    rel2_ref[0:NUM_RELROWS_K, HALF_K:DIM_K] = rip[:, 0:HALF_K]
    rel2_ref[NUM_RELROWS_K:2 * NUM_RELROWS_K, 0:HALF_K] = rr[:, HALF_K:DIM_K]
    rel2_ref[NUM_RELROWS_K:2 * NUM_RELROWS_K, HALF_K:DIM_K] = rip[:, HALF_K:DIM_K]
    rip_ref[...] = rip


def _node_body(agg_ref, deg_ref, rel_r_ref, rel_i_ref,
               we_ref, wr_ref, or_ref, oi_ref, ror_ref, roi_ref):
    deg = deg_ref[0:NUM_ENT_K, 0:1] + deg_ref[NUM_ENT_K:2 * NUM_ENT_K, 0:1]
    inv = jnp.where(deg == 0.0, 1.0, 1.0 / deg)
    w0 = we_ref[0:HALF_K, :]
    w1 = we_ref[HALF_K:DIM_K, :]
    ar0 = agg_ref[0:NUM_ENT_K, 0:HALF_K] * inv
    ai0 = agg_ref[0:NUM_ENT_K, HALF_K:DIM_K] * inv
    ar1 = agg_ref[NUM_ENT_K:2 * NUM_ENT_K, 0:HALF_K] * inv
    ai1 = agg_ref[NUM_ENT_K:2 * NUM_ENT_K, HALF_K:DIM_K] * inv
    or_ref[...] = jnp.tanh(jnp.dot(ar0, w0, preferred_element_type=_f32)
                           + jnp.dot(ar1, w1, preferred_element_type=_f32))
    oi_ref[...] = jnp.tanh(jnp.dot(ai0, w0, preferred_element_type=_f32)
                           + jnp.dot(ai1, w1, preferred_element_type=_f32))
    wr = wr_ref[...]
    ror_ref[...] = jnp.dot(rel_r_ref[...], wr, preferred_element_type=_f32)
    roi_ref[...] = jnp.dot(rel_i_ref[...], wr, preferred_element_type=_f32)


_MESH = plsc.VectorSubcoreMesh(core_axis_name="c", subcore_axis_name="s",
                               num_cores=NCORES, num_subcores=NSUB)


def _edge_body(ent2_hbm, rel2_hbm, src_hbm, dst_hbm, et_hbm,
               agg_out,
               isrc0, idst0, iet0, isrc1, idst1, iet1,
               hh0, rr0, hh1, rr1,
               zbuf, agg_sp, sem_g0, sem_g1):
    c = lax.axis_index("c")
    s = lax.axis_index("s")
    zero16 = jnp.zeros((LANES,), _f32)

    def _z_zbuf(e, carry):
        for j in range(DIM_K // LANES):
            zbuf[e, pl.ds(j * LANES, LANES)] = zero16
        return carry
    lax.fori_loop(0, BB, _z_zbuf, 0)

    base = s * STRIPE
    nb = jnp.where(s == NSUB - 1, (NUM_ENT_K - (NSUB - 1) * STRIPE) // BB,
                   STRIPE // BB)

    def _z_sp(b, carry):
        pltpu.sync_copy(zbuf, agg_sp.at[pl.ds(base + b * BB, BB)])
        return carry
    lax.fori_loop(0, nb, _z_sp, 0)
    plsc.subcore_barrier()

    ent_off = c * NUM_ENT_K
    rel_off = c * NUM_RELROWS_K

    def _issue(k, isrc, idst, iet, hh, rr, sem):
        kk = jnp.minimum(k, NQ - 1)
        eoff = (kk * NSUB + s) * CHUNK
        pltpu.sync_copy(src_hbm.at[pl.ds(eoff, CHUNK)], isrc)
        pltpu.sync_copy(dst_hbm.at[pl.ds(eoff, CHUNK)], idst)
        pltpu.sync_copy(et_hbm.at[pl.ds(eoff, CHUNK)], iet)

        def _shift(i, carry):
            sl = pl.ds(i * LANES, LANES)
            isrc[sl] = isrc[sl] + ent_off
            iet[sl] = iet[sl] + rel_off
            return carry
        lax.fori_loop(0, CHUNK // LANES, _shift, 0)
        pltpu.async_copy(ent2_hbm.at[isrc], hh, sem)
        pltpu.async_copy(rel2_hbm.at[iet], rr, sem)

    def _consume(isrc, idst, iet, hh, rr, sem):
        pltpu.make_async_copy(ent2_hbm.at[isrc], hh, sem).wait()
        pltpu.make_async_copy(rel2_hbm.at[iet], rr, sem).wait()

        def _row(e, carry):
            for j in range(HALF_K // LANES):
                sl = pl.ds(j * LANES, LANES)
                sh = pl.ds(HALF_K + j * LANES, LANES)
                hr = hh[e, sl]
                hi = hh[e, sh]
                rr_ = rr[e, sl]
                ri = rr[e, sh]
                hh[e, sl] = hr * rr_ - hi * ri
                hh[e, sh] = hr * ri + hi * rr_
            return carry
        lax.fori_loop(0, CHUNK, _row, 0)
        pltpu.sync_copy(hh, agg_sp.at[idst], add=True)

    _issue(0, isrc0, idst0, iet0, hh0, rr0, sem_g0)

    def _pair(p, carry):
        k = 2 * p
        _issue(k + 1, isrc1, idst1, iet1, hh1, rr1, sem_g1)
        _consume(isrc0, idst0, iet0, hh0, rr0, sem_g0)
        _issue(k + 2, isrc0, idst0, iet0, hh0, rr0, sem_g0)
        _consume(isrc1, idst1, iet1, hh1, rr1, sem_g1)
        return carry
    lax.fori_loop(0, NQ // 2, _pair, 0)
    pltpu.make_async_copy(ent2_hbm.at[isrc0], hh0, sem_g0).wait()
    pltpu.make_async_copy(rel2_hbm.at[iet0], rr0, sem_g0).wait()

    plsc.subcore_barrier()

    def _wb(b, carry):
        off = base + b * BB
        pltpu.sync_copy(agg_sp.at[pl.ds(off, BB)], zbuf)
        pltpu.sync_copy(zbuf, agg_out.at[pl.ds(ent_off + off, BB)])
        return carry
    lax.fori_loop(0, nb, _wb, 0)


_edge_kernel = functools.partial(
    pl.kernel,
    out_type=jax.ShapeDtypeStruct((2 * NUM_ENT_K, DIM_K), _f32),
    mesh=_MESH,
    scratch_types=[
        pltpu.VMEM((CHUNK,), _i32),
        pltpu.VMEM((CHUNK,), _i32),
        pltpu.VMEM((CHUNK,), _i32),
        pltpu.VMEM((CHUNK,), _i32),
        pltpu.VMEM((CHUNK,), _i32),
        pltpu.VMEM((CHUNK,), _i32),
        pltpu.VMEM((CHUNK, DIM_K), _f32),
        pltpu.VMEM((CHUNK, DIM_K), _f32),
        pltpu.VMEM((CHUNK, DIM_K), _f32),
        pltpu.VMEM((CHUNK, DIM_K), _f32),
        pltpu.VMEM((BB, DIM_K), _f32),
        pltpu.VMEM_SHARED((NUM_ENT_K, DIM_K), _f32),
        pltpu.SemaphoreType.DMA,
        pltpu.SemaphoreType.DMA,
    ],
    compiler_params=pltpu.CompilerParams(use_tc_tiling_on_sc=False),
)(_edge_body)


_EDGES_PER_CORE = N_EDGES_K // NCORES
_DCHUNK = 128
_DCHUNKS = _EDGES_PER_CORE // _DCHUNK
_DSTRIPE = 640
_DBB = 80


def _deg_body(dst_hbm, deg_out, idx_dst, ones_v, zdeg, deg_sp, sem0):
    c = lax.axis_index("c")
    s = lax.axis_index("s")
    zero16 = jnp.zeros((LANES,), _f32)
    one16 = jnp.ones((LANES,), _f32)

    def _fill_row(e, carry):
        ones_v[e, :] = one16
        return carry
    lax.fori_loop(0, _DCHUNK, _fill_row, 0)

    def _z_zdeg(e, carry):
        zdeg[e, :] = zero16
        return carry
    lax.fori_loop(0, _DBB, _z_zdeg, 0)

    base = s * _DSTRIPE
    nb = jnp.where(s == NSUB - 1, (NUM_ENT_K - (NSUB - 1) * _DSTRIPE) // _DBB,
                   _DSTRIPE // _DBB)

    def _z_sp(b, carry):
        pltpu.sync_copy(zdeg, deg_sp.at[pl.ds(base + b * _DBB, _DBB)])
        return carry
    lax.fori_loop(0, nb, _z_sp, 0)
    plsc.subcore_barrier()

    nq = jnp.where(s < _DCHUNKS - (_DCHUNKS // NSUB) * NSUB,
                   _DCHUNKS // NSUB + 1, _DCHUNKS // NSUB)

    def _chunk(q, carry):
        eoff = c * _EDGES_PER_CORE + (q * NSUB + s) * _DCHUNK
        pltpu.sync_copy(dst_hbm.at[pl.ds(eoff, _DCHUNK)], idx_dst)
        pltpu.sync_copy(ones_v, deg_sp.at[idx_dst], add=True)
        return carry
    lax.fori_loop(0, nq, _chunk, 0)

    plsc.subcore_barrier()

    def _wb(b, carry):
        off = base + b * _DBB
        pltpu.sync_copy(deg_sp.at[pl.ds(off, _DBB)], zdeg)
        pltpu.sync_copy(zdeg, deg_out.at[pl.ds(c * NUM_ENT_K + off, _DBB)])
        return carry
    lax.fori_loop(0, nb, _wb, 0)


_deg_kernel = functools.partial(
    pl.kernel,
    out_type=jax.ShapeDtypeStruct((2 * NUM_ENT_K, LANES), _f32),
    mesh=_MESH,
    scratch_types=[
        pltpu.VMEM((_DCHUNK,), _i32),
        pltpu.VMEM((_DCHUNK, LANES), _f32),
        pltpu.VMEM((_DBB, LANES), _f32),
        pltpu.VMEM_SHARED((NUM_ENT_K, LANES), _f32),
        pltpu.SemaphoreType.DMA,
    ],
    compiler_params=pltpu.CompilerParams(use_tc_tiling_on_sc=False),
)(_deg_body)


def _gather_body(out_r_hbm, out_i_hbm, ror_hbm, roi_hbm, sub_hbm, rel_hbm,
                 ser_out, sei_out, rer_out, rei_out,
                 idx_v, buf, sem):
    c = lax.axis_index("c")
    s = lax.axis_index("s")
    wid = s * NCORES + c
    per = BATCH_K // (NCORES * NSUB)
    base = wid * per
    pltpu.sync_copy(sub_hbm.at[pl.ds(base, per)], idx_v)
    pltpu.async_copy(out_r_hbm.at[idx_v], buf, sem).wait()
    pltpu.sync_copy(buf, ser_out.at[pl.ds(base, per)])
    pltpu.async_copy(out_i_hbm.at[idx_v], buf, sem).wait()
    pltpu.sync_copy(buf, sei_out.at[pl.ds(base, per)])
    pltpu.sync_copy(rel_hbm.at[pl.ds(base, per)], idx_v)
    pltpu.async_copy(ror_hbm.at[idx_v], buf, sem).wait()
    pltpu.sync_copy(buf, rer_out.at[pl.ds(base, per)])
    pltpu.async_copy(roi_hbm.at[idx_v], buf, sem).wait()
    pltpu.sync_copy(buf, rei_out.at[pl.ds(base, per)])


_gather_kernel = functools.partial(
    pl.kernel,
    out_type=(
        jax.ShapeDtypeStruct((BATCH_K, DIM_K), _f32),
        jax.ShapeDtypeStruct((BATCH_K, DIM_K), _f32),
        jax.ShapeDtypeStruct((BATCH_K, DIM_K), _f32),
        jax.ShapeDtypeStruct((BATCH_K, DIM_K), _f32),
    ),
    mesh=_MESH,
    scratch_types=[
        pltpu.VMEM((BATCH_K // (NCORES * NSUB),), _i32),
        pltpu.VMEM((BATCH_K // (NCORES * NSUB), DIM_K), _f32),
        pltpu.SemaphoreType.DMA,
    ],
)(_gather_body)


def kernel(init_embed_real, init_embed_imag, init_rel_real, init_rel_imag,
           im_proj, W_ent, W_rel, edge_index, edge_type, sub, rel):
    ent2, rel2, rel_i = pl.pallas_call(
        _proj_body,
        out_shape=(
            jax.ShapeDtypeStruct((2 * NUM_ENT_K, DIM_K), _f32),
            jax.ShapeDtypeStruct((2 * NUM_RELROWS_K, DIM_K), _f32),
            jax.ShapeDtypeStruct((NUM_RELROWS_K, DIM_K), _f32),
        ),
    )(init_embed_real, init_embed_imag, init_rel_real, init_rel_imag, im_proj)

    src = edge_index[0].astype(_i32)
    dst = edge_index[1].astype(_i32)
    et = edge_type.astype(_i32)

    deg16 = _deg_kernel(dst)
    agg2 = _edge_kernel(ent2, rel2, src, dst, et)

    out_r, out_i, rel_out_r, rel_out_i = pl.pallas_call(
        _node_body,
        out_shape=(
            jax.ShapeDtypeStruct((NUM_ENT_K, DIM_K), _f32),
            jax.ShapeDtypeStruct((NUM_ENT_K, DIM_K), _f32),
            jax.ShapeDtypeStruct((NUM_RELROWS_K, DIM_K), _f32),
            jax.ShapeDtypeStruct((NUM_RELROWS_K, DIM_K), _f32),
        ),
    )(agg2, deg16, init_rel_real, rel_i, W_ent, W_rel)

    sub_emb_r, sub_emb_i, rel_emb_r, rel_emb_i = _gather_kernel(
        out_r, out_i, rel_out_r, rel_out_i,
        sub.astype(_i32), rel.astype(_i32))

    return (sub_emb_r, sub_emb_i, rel_emb_r, rel_emb_i, out_r, out_i)
